# baseline (speedup 1.0000x reference)
import functools

import jax
import jax.numpy as jnp
from jax import lax
from jax.experimental import pallas as pl
from jax.experimental.pallas import tpu as pltpu
from jax.experimental.pallas import tpu_sc as plsc

_DIM = 1024
_ROWS = 8192
_NC, _NS = 2, 16
_ROWS_PER_SC = _ROWS // _NC
_CHUNK = 512
_NCHUNK = _ROWS_PER_SC // _CHUNK
_NBUF = 2


@functools.partial(
    pl.kernel,
    mesh=plsc.VectorSubcoreMesh(core_axis_name="c", subcore_axis_name="s"),
    out_type=jax.ShapeDtypeStruct((_ROWS, _DIM), jnp.float32),
    scratch_types=(
        [pltpu.VMEM_SHARED((_CHUNK, _DIM), jnp.float32) for _ in range(_NBUF)]
        + [pltpu.SemaphoreType.DMA for _ in range(2 * _NBUF)]
    ),
)
def _sc_copy(emb_hbm, out_hbm, *scratch):
    bufs = scratch[:_NBUF]
    gsems = scratch[_NBUF:2 * _NBUF]
    ssems = scratch[2 * _NBUF:]
    cid = lax.axis_index("c")
    sid = lax.axis_index("s")
    base = cid * _ROWS_PER_SC

    def gather(i):
        b = i % _NBUF
        return pltpu.make_async_copy(
            emb_hbm.at[pl.ds(base + i * _CHUNK, _CHUNK)], bufs[b], gsems[b])

    def scatter(i):
        b = i % _NBUF
        return pltpu.make_async_copy(
            bufs[b], out_hbm.at[pl.ds(base + i * _CHUNK, _CHUNK)], ssems[b])

    @pl.when(sid == 0)
    def _():
        gather(0).start()
        for i in range(_NCHUNK):
            gather(i).wait()
            scatter(i).start()
            if i + 1 < _NCHUNK:
                if i >= 1:
                    scatter(i - 1).wait()
                gather(i + 1).start()
        scatter(_NCHUNK - 2).wait()
        scatter(_NCHUNK - 1).wait()


def kernel(x, emb_weight):
    del x
    return _sc_copy(emb_weight)[None, :, :]
